# Initial kernel scaffold; baseline (speedup 1.0000x reference)
import jax
import jax.numpy as jnp
from jax.experimental import pallas as pl


def kernel(encodings, subnetwork, W, b):
    raise NotImplementedError("write your pallas kernel here")


# speedup vs baseline: 8.3232x; 8.3232x over previous
import jax
import jax.numpy as jnp
from jax import lax
from jax.experimental import pallas as pl
from jax.experimental.pallas import tpu as pltpu
from jax.experimental.pallas import tpu_sc as plsc

_N = 10000
_E = 320000
_D = 128
_NC = 2
_NS = 16
_NW = _NC * _NS
_EPW = _E // _NW
_C = 80
_CH = _EPW // _C
_ZR = 80
_ZCH = _N // _ZR
_ZPT = 8


def _sc_body(x_hbm, src_hbm, dst_hbm, acc_hbm, deg_hbm, src_v, dst_v, msg_v,
             zbuf_v, ones_v, zdeg_v, acc_sp, deg_sp, sem):
    cid = lax.axis_index("c")
    sid = lax.axis_index("s")
    wid = sid * _NC + cid

    pltpu.sync_copy(src_hbm.at[pl.ds(wid * _EPW, _EPW)], src_v)
    pltpu.sync_copy(dst_hbm.at[wid], dst_v)

    @pl.loop(0, _ZR)
    def _zrow(r):
        @pl.loop(0, _D // 16)
        def _zcol(c):
            zbuf_v[r, pl.ds(c * 16, 16)] = jnp.zeros((16,), jnp.float32)

    @pl.loop(0, _C // 16)
    def _fill(i):
        ones_v[pl.ds(i * 16, 16)] = jnp.ones((16,), jnp.float32)
        zdeg_v[pl.ds(i * 16, 16)] = jnp.zeros((16,), jnp.float32)

    @pl.loop(0, _ZPT)
    def _zcopy(k):
        ck = sid * _ZPT + k

        @pl.when(ck < _ZCH)
        def _():
            pltpu.sync_copy(zbuf_v, acc_sp.at[pl.ds(ck * _ZR, _ZR)])
            pltpu.sync_copy(zdeg_v, deg_sp.at[pl.ds(ck * _ZR, _ZR)])

    plsc.subcore_barrier()

    @pl.loop(0, _CH)
    def _step(j):
        pltpu.async_copy(x_hbm.at[src_v.at[pl.ds(j * _C, _C)]], msg_v,
                         sem).wait()
        pltpu.sync_copy(msg_v, acc_sp.at[dst_v.at[j]], add=True)
        pltpu.sync_copy(ones_v, deg_sp.at[dst_v.at[j]], add=True)

    plsc.subcore_barrier()

    @pl.loop(0, _ZPT)
    def _ocopy(k):
        ck = sid * _ZPT + k

        @pl.when(ck < _ZCH)
        def _():
            pltpu.sync_copy(acc_sp.at[pl.ds(ck * _ZR, _ZR)],
                            acc_hbm.at[cid, pl.ds(ck * _ZR, _ZR)])
            pltpu.sync_copy(deg_sp.at[pl.ds(ck * _ZR, _ZR)], zdeg_v)
            pltpu.sync_copy(zdeg_v,
                            deg_hbm.at[pl.ds(cid * _N + ck * _ZR, _ZR)])


def _sc_aggregate(x, src, dst3d):
    mesh = plsc.VectorSubcoreMesh(core_axis_name="c", subcore_axis_name="s")
    f = pl.kernel(
        _sc_body,
        out_type=[
            jax.ShapeDtypeStruct((_NC, _N, _D), jnp.float32),
            jax.ShapeDtypeStruct((_NC * _N,), jnp.float32),
        ],
        mesh=mesh,
        scratch_types=[
            pltpu.VMEM((_EPW,), jnp.int32),
            pltpu.VMEM((_CH, _C), jnp.int32),
            pltpu.VMEM((_C, _D), jnp.float32),
            pltpu.VMEM((_ZR, _D), jnp.float32),
            pltpu.VMEM((_C,), jnp.float32),
            pltpu.VMEM((_ZR,), jnp.float32),
            pltpu.VMEM_SHARED((_N, _D), jnp.float32),
            pltpu.VMEM_SHARED((_N,), jnp.float32),
            pltpu.SemaphoreType.DMA,
        ],
    )
    return f(x, src, dst3d)


def _tc_body(x_ref, a_ref, d_ref, w_ref, b_ref, o_ref):
    a = a_ref[0] + a_ref[1]
    deg = jnp.maximum(d_ref[0] + d_ref[1], 1.0)
    mean = a / deg
    o_ref[...] = (
        jnp.dot(x_ref[...], w_ref[0], preferred_element_type=jnp.float32)
        + jnp.dot(mean, w_ref[1], preferred_element_type=jnp.float32)
        + b_ref[...])


def _tc_combine(x, accp, degp, W2, b2):
    bm = 1000
    return pl.pallas_call(
        _tc_body,
        grid=(_N // bm,),
        in_specs=[
            pl.BlockSpec((bm, _D), lambda i: (i, 0)),
            pl.BlockSpec((_NC, bm, _D), lambda i: (0, i, 0)),
            pl.BlockSpec((_NC, bm, 1), lambda i: (0, i, 0)),
            pl.BlockSpec((2, _D, _D), lambda i: (0, 0, 0)),
            pl.BlockSpec((1, _D), lambda i: (0, 0)),
        ],
        out_specs=pl.BlockSpec((bm, _D), lambda i: (i, 0)),
        out_shape=jax.ShapeDtypeStruct((_N, _D), jnp.float32),
    )(x, accp, degp, W2, b2)


@jax.jit
def kernel(encodings, subnetwork, W, b):
    src = subnetwork[0].astype(jnp.int32)
    dst = subnetwork[1].astype(jnp.int32)
    dst3d = dst.reshape(_NW, _CH, _C)
    accp, degp = _sc_aggregate(encodings, src, dst3d)
    return _tc_combine(encodings, accp, degp.reshape(_NC, _N, 1),
                       W.reshape(2, _D, _D), b.reshape(1, _D))


# speedup vs baseline: 12.5328x; 1.5058x over previous
import jax
import jax.numpy as jnp
from jax import lax
from jax.experimental import pallas as pl
from jax.experimental.pallas import tpu as pltpu
from jax.experimental.pallas import tpu_sc as plsc

_N = 10000
_E = 320000
_D = 128
_NC = 2
_NS = 16
_NW = _NC * _NS
_EPW = _E // _NW
_C = 80
_CH = _EPW // _C
_ZR = 80
_ZB = 16
_ZCH = _N // _ZR
_ZPT = 8


def _sc_body(x_hbm, src_hbm, dst_hbm, acc_hbm, deg_hbm, src_v, dst_v, msg0_v,
             msg1_v, zbuf_v, ones_v, zdeg_v, acc_sp, deg_sp, sem0, sem1):
    cid = lax.axis_index("c")
    sid = lax.axis_index("s")
    wid = sid * _NC + cid

    pltpu.sync_copy(src_hbm.at[pl.ds(wid * _EPW, _EPW)], src_v)
    pltpu.sync_copy(dst_hbm.at[wid], dst_v)

    @pl.loop(0, _ZB)
    def _zrow(r):
        @pl.loop(0, _D // 16)
        def _zcol(c):
            zbuf_v[r, pl.ds(c * 16, 16)] = jnp.zeros((16,), jnp.float32)

    @pl.loop(0, _C // 16)
    def _fill(i):
        ones_v[pl.ds(i * 16, 16)] = jnp.ones((16,), jnp.float32)
        zdeg_v[pl.ds(i * 16, 16)] = jnp.zeros((16,), jnp.float32)

    @pl.loop(0, _ZPT)
    def _zcopy(k):
        ck = sid * _ZPT + k

        @pl.when(ck < _ZCH)
        def _():
            @pl.loop(0, _ZR // _ZB)
            def _zsub(m):
                pltpu.sync_copy(
                    zbuf_v, acc_sp.at[pl.ds(ck * _ZR + m * _ZB, _ZB)])

            pltpu.sync_copy(zdeg_v, deg_sp.at[pl.ds(ck * _ZR, _ZR)])

    plsc.subcore_barrier()

    def _gather(chunk, buf, sem):
        return pltpu.async_copy(
            x_hbm.at[src_v.at[pl.ds(chunk * _C, _C)]], buf, sem)

    _gather(0, msg0_v, sem0)
    _gather(1, msg1_v, sem1)

    @pl.loop(0, _CH, step=2)
    def _step(j):
        for b, (buf, sem) in enumerate(((msg0_v, sem0), (msg1_v, sem1))):
            chunk = j + b

            @pl.when(chunk < _CH)
            def _():
                pltpu.make_async_copy(
                    x_hbm.at[src_v.at[pl.ds(chunk * _C, _C)]], buf,
                    sem).wait()
                pltpu.sync_copy(buf, acc_sp.at[dst_v.at[chunk]], add=True)
                pltpu.sync_copy(ones_v, deg_sp.at[dst_v.at[chunk]], add=True)

                @pl.when(chunk + 2 < _CH)
                def _():
                    _gather(chunk + 2, buf, sem)

    plsc.subcore_barrier()

    @pl.loop(0, _ZPT)
    def _ocopy(k):
        ck = sid * _ZPT + k

        @pl.when(ck < _ZCH)
        def _():
            pltpu.sync_copy(acc_sp.at[pl.ds(ck * _ZR, _ZR)],
                            acc_hbm.at[cid, pl.ds(ck * _ZR, _ZR)])
            pltpu.sync_copy(deg_sp.at[pl.ds(ck * _ZR, _ZR)], zdeg_v)
            pltpu.sync_copy(zdeg_v,
                            deg_hbm.at[pl.ds(cid * _N + ck * _ZR, _ZR)])


def _sc_aggregate(x, src, dst3d):
    mesh = plsc.VectorSubcoreMesh(core_axis_name="c", subcore_axis_name="s")
    f = pl.kernel(
        _sc_body,
        out_type=[
            jax.ShapeDtypeStruct((_NC, _N, _D), jnp.float32),
            jax.ShapeDtypeStruct((_NC * _N,), jnp.float32),
        ],
        mesh=mesh,
        scratch_types=[
            pltpu.VMEM((_EPW,), jnp.int32),
            pltpu.VMEM((_CH, _C), jnp.int32),
            pltpu.VMEM((_C, _D), jnp.float32),
            pltpu.VMEM((_C, _D), jnp.float32),
            pltpu.VMEM((_ZB, _D), jnp.float32),
            pltpu.VMEM((_C,), jnp.float32),
            pltpu.VMEM((_ZR,), jnp.float32),
            pltpu.VMEM_SHARED((_N, _D), jnp.float32),
            pltpu.VMEM_SHARED((_N,), jnp.float32),
            pltpu.SemaphoreType.DMA,
            pltpu.SemaphoreType.DMA,
        ],
    )
    return f(x, src, dst3d)


def _tc_body(x_ref, a_ref, d_ref, w_ref, b_ref, o_ref):
    a = a_ref[0] + a_ref[1]
    deg = jnp.maximum(d_ref[0] + d_ref[1], 1.0)
    mean = a / deg
    o_ref[...] = (
        jnp.dot(x_ref[...], w_ref[0], preferred_element_type=jnp.float32)
        + jnp.dot(mean, w_ref[1], preferred_element_type=jnp.float32)
        + b_ref[...])


def _tc_combine(x, accp, degp, W2, b2):
    bm = 1000
    return pl.pallas_call(
        _tc_body,
        grid=(_N // bm,),
        in_specs=[
            pl.BlockSpec((bm, _D), lambda i: (i, 0)),
            pl.BlockSpec((_NC, bm, _D), lambda i: (0, i, 0)),
            pl.BlockSpec((_NC, bm, 1), lambda i: (0, i, 0)),
            pl.BlockSpec((2, _D, _D), lambda i: (0, 0, 0)),
            pl.BlockSpec((1, _D), lambda i: (0, 0)),
        ],
        out_specs=pl.BlockSpec((bm, _D), lambda i: (i, 0)),
        out_shape=jax.ShapeDtypeStruct((_N, _D), jnp.float32),
    )(x, accp, degp, W2, b2)


@jax.jit
def kernel(encodings, subnetwork, W, b):
    src = subnetwork[0].astype(jnp.int32)
    dst = subnetwork[1].astype(jnp.int32)
    dst3d = dst.reshape(_NW, _CH, _C)
    accp, degp = _sc_aggregate(encodings, src, dst3d)
    return _tc_combine(encodings, accp, degp.reshape(_NC, _N, 1),
                       W.reshape(2, _D, _D), b.reshape(1, _D))
